# baseline (speedup 1.0000x reference)
import functools

import jax
import jax.numpy as jnp
from jax import lax
from jax.experimental import pallas as pl
from jax.experimental.pallas import tpu as pltpu
from jax.experimental.pallas import tpu_sc as plsc

N, E, C = 10000, 320000, 128
NP = 10240
NC, NS, L = 2, 16, 16
NW = NC * NS
EPW = E // NW
KE = 80
NCH = EPW // KE
RPT = NP // NS
CCH = C // L

_sc_mesh = plsc.VectorSubcoreMesh(core_axis_name="c", subcore_axis_name="s")


@functools.partial(
    pl.kernel,
    out_type=(
        jax.ShapeDtypeStruct((NC, NP, C), jnp.float32),
        jax.ShapeDtypeStruct((NC, NP), jnp.float32),
    ),
    mesh=_sc_mesh,
    compiler_params=pltpu.CompilerParams(use_tc_tiling_on_sc=False),
    scratch_types=(
        pltpu.VMEM((C,), jnp.float32),
        pltpu.VMEM((C,), jnp.float32),
        pltpu.VMEM((KE, C), jnp.float32),
        pltpu.VMEM((KE, C), jnp.float32),
        pltpu.VMEM((KE, C), jnp.float32),
        pltpu.VMEM((KE, L), jnp.float32),
        pltpu.VMEM((KE, L), jnp.float32),
        pltpu.VMEM((KE, L), jnp.float32),
        pltpu.VMEM((KE, L), jnp.float32),
        pltpu.VMEM((KE, L), jnp.float32),
        pltpu.VMEM((KE, L), jnp.float32),
        pltpu.VMEM((1, KE), jnp.int32),
        pltpu.VMEM((1, KE), jnp.int32),
        pltpu.VMEM((1, KE), jnp.int32),
        pltpu.VMEM((1, KE), jnp.int32),
        pltpu.VMEM((1, KE), jnp.int32),
        pltpu.VMEM((1, KE), jnp.int32),
        pltpu.VMEM((1, KE), jnp.int32),
        pltpu.VMEM((1, KE), jnp.int32),
        pltpu.VMEM((1, KE), jnp.int32),
        pltpu.VMEM((KE,), jnp.float32),
        pltpu.VMEM((RPT,), jnp.float32),
        pltpu.VMEM_SHARED((NP, C), jnp.float32),
        pltpu.VMEM_SHARED((NP,), jnp.float32),
        pltpu.SemaphoreType.DMA,
        pltpu.SemaphoreType.DMA,
        pltpu.SemaphoreType.DMA,
        pltpu.SemaphoreType.DMA,
        pltpu.SemaphoreType.DMA,
        pltpu.SemaphoreType.DMA,
        pltpu.SemaphoreType.DMA,
        pltpu.SemaphoreType.DMA,
        pltpu.SemaphoreType.DMA,
    ),
)
def _sc_agg(h_hbm, pk_hbm, s2_hbm, w_hbm, b_hbm, p_hbm, cnt_hbm,
            w_v, b_v, g0, g1, g2, gs0, gs1, gs2, gd0, gd1, gd2,
            pk0, pk1, pk2, sb0, sb1, sb2, db0, db1, db2, ones_v, cnt_v,
            acc_sh, cnt_sh,
            gsem0, gsem1, gsem2, ssem0, ssem1, ssem2, psem0, psem1, psem2):
    cid = lax.axis_index("c")
    sid = lax.axis_index("s")
    wid = sid * NC + cid

    pltpu.sync_copy(w_hbm, w_v)
    pltpu.sync_copy(b_hbm, b_v)

    zero16 = jnp.zeros((L,), jnp.float32)
    one16 = jnp.ones((L,), jnp.float32)

    def _zrow(i, _):
        for c in range(CCH):
            g0[i, pl.ds(c * L, L)] = zero16
        return 0
    lax.fori_loop(0, KE, _zrow, 0)

    def _zcnt(i, _):
        cnt_v[pl.ds(i * L, L)] = zero16
        return 0
    lax.fori_loop(0, RPT // L, _zcnt, 0)
    for q in range(KE // L):
        ones_v[pl.ds(q * L, L)] = one16

    row0 = sid * RPT
    for k in range(RPT // KE):
        pltpu.sync_copy(g0, acc_sh.at[pl.ds(row0 + k * KE, KE)])
    pltpu.sync_copy(cnt_v, cnt_sh.at[pl.ds(row0, RPT)])

    wv = [w_v[pl.ds(c * L, L)] for c in range(CCH)]
    bv = [b_v[pl.ds(c * L, L)] for c in range(CCH)]

    plsc.subcore_barrier()

    S = ((g0, gs0, gd0, pk0, sb0, db0, gsem0, ssem0, psem0),
         (g1, gs1, gd1, pk1, sb1, db1, gsem1, ssem1, psem1),
         (g2, gs2, gd2, pk2, sb2, db2, gsem2, ssem2, psem2))

    def _prep_pk(jn, slot):
        pk = slot[3]
        pltpu.async_copy(pk_hbm.at[wid, pl.ds(jn, 1)], pk, slot[8])

    def _prep_gather(jn, slot):
        g, gs, gd, pk, sb, db, gsem, _, psem = slot
        pltpu.make_async_copy(pk_hbm.at[wid, pl.ds(jn, 1)], pk, psem).wait()
        for q in range(KE // L):
            pkv = pk[0, pl.ds(q * L, L)]
            sb[0, pl.ds(q * L, L)] = lax.bitwise_and(pkv, 16383)
            db[0, pl.ds(q * L, L)] = lax.shift_right_logical(pkv, 14)
        pltpu.async_copy(h_hbm.at[sb.at[0]], g, gsem)
        pltpu.async_copy(s2_hbm.at[sb.at[0]], gs, gsem)
        pltpu.async_copy(s2_hbm.at[db.at[0]], gd, gsem)

    def _drain(slot):
        g, db, ssem = slot[0], slot[5], slot[7]
        pltpu.make_async_copy(g, acc_sh.at[db.at[0]], ssem).wait()
        pltpu.make_async_copy(ones_v, cnt_sh.at[db.at[0]], ssem).wait()

    def _process(slot):
        g, gs, gd, pk, sb, db, gsem, ssem, _ = slot
        pltpu.make_async_copy(h_hbm.at[sb.at[0]], g, gsem).wait()
        pltpu.make_async_copy(s2_hbm.at[sb.at[0]], gs, gsem).wait()
        pltpu.make_async_copy(s2_hbm.at[db.at[0]], gd, gsem).wait()

        @plsc.parallel_loop(0, KE, 1, unroll=8)
        def _edge(e):
            t = gs[e, pl.ds(0, L)] + gd[e, pl.ds(0, L)]
            for c in range(CCH):
                gv = g[e, pl.ds(c * L, L)]
                g[e, pl.ds(c * L, L)] = gv + jnp.maximum(t * wv[c] + bv[c], 0.0)

        pltpu.async_copy(g, acc_sh.at[db.at[0]], ssem, add=True)
        pltpu.async_copy(ones_v, cnt_sh.at[db.at[0]], ssem, add=True)

    _prep_pk(0, S[0])
    _prep_pk(1, S[1])
    _prep_pk(2, S[2])
    _prep_gather(0, S[0])
    _prep_gather(1, S[1])

    def _step(st, _):
        for b in range(3):
            j = 3 * st + b
            nxt = S[(b + 2) % 3]
            cur = S[b]

            @pl.when(j >= 1)
            def _():
                _drain(nxt)

            @pl.when(j + 2 <= NCH - 1)
            def _():
                _prep_gather(j + 2, nxt)

            @pl.when(j + 3 <= NCH - 1)
            def _():
                _prep_pk(j + 3, cur)

            @pl.when(j <= NCH - 1)
            def _():
                _process(cur)
        return 0
    lax.fori_loop(0, (NCH + 3) // 3, _step, 0)

    plsc.subcore_barrier()

    for k in range(RPT // KE):
        pltpu.sync_copy(acc_sh.at[pl.ds(row0 + k * KE, KE)], g0)
        pltpu.sync_copy(g0, p_hbm.at[cid, pl.ds(row0 + k * KE, KE)])
    pltpu.sync_copy(cnt_sh.at[pl.ds(row0, RPT)], cnt_v)
    pltpu.sync_copy(cnt_v, cnt_hbm.at[cid, pl.ds(row0, RPT)])


def _tc_r():
    BN = 1024

    def body(h_ref, wr_ref, bl_ref, o_ref):
        o_ref[...] = (jnp.dot(h_ref[...], wr_ref[...],
                              preferred_element_type=jnp.float32) + bl_ref[...])

    return pl.pallas_call(
        body,
        grid=(NP // BN,),
        in_specs=[
            pl.BlockSpec((BN, C), lambda i: (i, 0)),
            pl.BlockSpec((C, C), lambda i: (0, 0)),
            pl.BlockSpec((1, C), lambda i: (0, 0)),
        ],
        out_specs=pl.BlockSpec((BN, C), lambda i: (i, 0)),
        out_shape=jax.ShapeDtypeStruct((NP, C), jnp.float32),
    )


def _tc_post(relu_out):
    BN = 1024

    def body(p_ref, cnt_ref, hr_ref, wl_ref, o_ref):
        summ = p_ref[0] + p_ref[1]
        c = cnt_ref[:, 0:1] + cnt_ref[:, 1:2]
        agg = summ / jnp.maximum(c, 1.0)
        out = (jnp.dot(agg, wl_ref[...], preferred_element_type=jnp.float32)
               + hr_ref[...])
        nrm = jnp.sqrt(jnp.sum(out * out, axis=-1, keepdims=True))
        out = out / jnp.maximum(nrm, 1e-12)
        if relu_out:
            out = jnp.maximum(out, 0.0)
        o_ref[...] = out

    return pl.pallas_call(
        body,
        grid=(NP // BN,),
        in_specs=[
            pl.BlockSpec((2, BN, C), lambda i: (0, i, 0)),
            pl.BlockSpec((BN, 2), lambda i: (i, 0)),
            pl.BlockSpec((BN, C), lambda i: (i, 0)),
            pl.BlockSpec((C, C), lambda i: (0, 0)),
        ],
        out_specs=pl.BlockSpec((BN, C), lambda i: (i, 0)),
        out_shape=jax.ShapeDtypeStruct((NP, C), jnp.float32),
    )


_tc_right = _tc_r()
_tc_layer = _tc_post(True)
_tc_final = _tc_post(False)


def kernel(x, edge_index, spd, W_e0, b_e0, W_l0, b_l0, W_r0,
           W_e1, b_e1, W_l1, b_l1, W_r1, W_e2, b_e2, W_l2, b_l2, W_r2):
    s = jnp.mean(spd, axis=1)
    s2 = jnp.broadcast_to(s[:, None], (N, L))
    packed = (edge_index[0] | (edge_index[1] << 14)).reshape(NW, NCH, KE)

    h = jnp.pad(x, ((0, NP - N), (0, 0)))
    layers = ((W_e0, b_e0, W_l0, b_l0, W_r0, True),
              (W_e1, b_e1, W_l1, b_l1, W_r1, True),
              (W_e2, b_e2, W_l2, b_l2, W_r2, False))
    for We, be, Wl, bl, Wr, inner in layers:
        p, cnt = _sc_agg(h, packed, s2, We[:, 0], 2.0 * be)
        hr = _tc_right(h, Wr.T, bl[None])
        tc = _tc_layer if inner else _tc_final
        h = tc(p, cnt.T, hr, Wl.T)
    return h[:N]


# speedup vs baseline: 2.6572x; 2.6572x over previous
import functools

import jax
import jax.numpy as jnp
from jax import lax
from jax.experimental import pallas as pl
from jax.experimental.pallas import tpu as pltpu
from jax.experimental.pallas import tpu_sc as plsc

N, E, C = 10000, 320000, 128
NP = 10240
NC, NS, L = 2, 16, 16
NW = NC * NS
EPW = E // NW
KE = 80
NCH = EPW // KE
RPT = NP // NS
CCH = C // L

_sc_mesh = plsc.VectorSubcoreMesh(core_axis_name="c", subcore_axis_name="s")


@functools.partial(
    pl.kernel,
    out_type=(
        jax.ShapeDtypeStruct((NC, NP, C), jnp.float32),
        jax.ShapeDtypeStruct((NC, NP), jnp.float32),
    ),
    mesh=_sc_mesh,
    compiler_params=pltpu.CompilerParams(use_tc_tiling_on_sc=False),
    scratch_types=(
        pltpu.VMEM((C,), jnp.float32),
        pltpu.VMEM((C,), jnp.float32),
        pltpu.VMEM((KE, C), jnp.float32),
        pltpu.VMEM((KE, C), jnp.float32),
        pltpu.VMEM((KE, C), jnp.float32),
        pltpu.VMEM((KE, L), jnp.float32),
        pltpu.VMEM((KE, L), jnp.float32),
        pltpu.VMEM((KE, L), jnp.float32),
        pltpu.VMEM((KE, L), jnp.float32),
        pltpu.VMEM((KE, L), jnp.float32),
        pltpu.VMEM((KE, L), jnp.float32),
        pltpu.VMEM((1, KE), jnp.int32),
        pltpu.VMEM((1, KE), jnp.int32),
        pltpu.VMEM((1, KE), jnp.int32),
        pltpu.VMEM((1, KE), jnp.int32),
        pltpu.VMEM((1, KE), jnp.int32),
        pltpu.VMEM((1, KE), jnp.int32),
        pltpu.VMEM((1, KE), jnp.int32),
        pltpu.VMEM((1, KE), jnp.int32),
        pltpu.VMEM((1, KE), jnp.int32),
        pltpu.VMEM((KE,), jnp.float32),
        pltpu.VMEM((RPT,), jnp.float32),
        pltpu.VMEM_SHARED((NP, C), jnp.float32),
        pltpu.VMEM_SHARED((NP,), jnp.float32),
        pltpu.SemaphoreType.DMA,
        pltpu.SemaphoreType.DMA,
        pltpu.SemaphoreType.DMA,
        pltpu.SemaphoreType.DMA,
        pltpu.SemaphoreType.DMA,
        pltpu.SemaphoreType.DMA,
        pltpu.SemaphoreType.DMA,
        pltpu.SemaphoreType.DMA,
        pltpu.SemaphoreType.DMA,
    ),
)
def _sc_agg(h_hbm, pk_hbm, s2_hbm, w_hbm, b_hbm, p_hbm, cnt_hbm,
            w_v, b_v, g0, g1, g2, gs0, gs1, gs2, gd0, gd1, gd2,
            pk0, pk1, pk2, sb0, sb1, sb2, db0, db1, db2, ones_v, cnt_v,
            acc_sh, cnt_sh,
            gsem0, gsem1, gsem2, ssem0, ssem1, ssem2, psem0, psem1, psem2):
    cid = lax.axis_index("c")
    sid = lax.axis_index("s")
    wid = sid * NC + cid

    pltpu.sync_copy(w_hbm, w_v)
    pltpu.sync_copy(b_hbm, b_v)

    zero16 = jnp.zeros((L,), jnp.float32)
    one16 = jnp.ones((L,), jnp.float32)

    def _zrow(i, _):
        for c in range(CCH):
            g0[i, pl.ds(c * L, L)] = zero16
        return 0
    lax.fori_loop(0, KE, _zrow, 0)

    def _zcnt(i, _):
        cnt_v[pl.ds(i * L, L)] = zero16
        return 0
    lax.fori_loop(0, RPT // L, _zcnt, 0)
    for q in range(KE // L):
        ones_v[pl.ds(q * L, L)] = one16

    row0 = sid * RPT
    for k in range(RPT // KE):
        pltpu.sync_copy(g0, acc_sh.at[pl.ds(row0 + k * KE, KE)])
    pltpu.sync_copy(cnt_v, cnt_sh.at[pl.ds(row0, RPT)])

    wv = [w_v[pl.ds(c * L, L)] for c in range(CCH)]
    bv = [b_v[pl.ds(c * L, L)] for c in range(CCH)]

    plsc.subcore_barrier()

    S = ((g0, gs0, gd0, pk0, sb0, db0, gsem0, ssem0, psem0),
         (g1, gs1, gd1, pk1, sb1, db1, gsem1, ssem1, psem1),
         (g2, gs2, gd2, pk2, sb2, db2, gsem2, ssem2, psem2))

    def _prep_pk(jn, slot):
        pk = slot[3]
        pltpu.async_copy(pk_hbm.at[wid, pl.ds(jn, 1)], pk, slot[8])

    def _prep_gather(jn, slot):
        g, gs, gd, pk, sb, db, gsem, _, psem = slot
        pltpu.make_async_copy(pk_hbm.at[wid, pl.ds(jn, 1)], pk, psem).wait()
        for q in range(KE // L):
            pkv = pk[0, pl.ds(q * L, L)]
            sb[0, pl.ds(q * L, L)] = lax.bitwise_and(pkv, 16383)
            db[0, pl.ds(q * L, L)] = lax.shift_right_logical(pkv, 14)
        pltpu.async_copy(h_hbm.at[sb.at[0]], g, gsem)
        pltpu.async_copy(s2_hbm.at[sb.at[0]], gs, gsem)
        pltpu.async_copy(s2_hbm.at[db.at[0]], gd, gsem)

    def _drain(slot):
        g, db, ssem = slot[0], slot[5], slot[7]
        pltpu.make_async_copy(g, acc_sh.at[db.at[0]], ssem).wait()
        pltpu.make_async_copy(ones_v, cnt_sh.at[db.at[0]], ssem).wait()

    def _process(slot):
        g, gs, gd, pk, sb, db, gsem, ssem, _ = slot
        pltpu.make_async_copy(h_hbm.at[sb.at[0]], g, gsem).wait()
        pltpu.make_async_copy(s2_hbm.at[sb.at[0]], gs, gsem).wait()
        pltpu.make_async_copy(s2_hbm.at[db.at[0]], gd, gsem).wait()

        @plsc.parallel_loop(0, KE, 1, unroll=4)
        def _edge(e):
            t = gs[e, pl.ds(0, L)] + gd[e, pl.ds(0, L)]
            for c in range(CCH):
                gv = g[e, pl.ds(c * L, L)]
                g[e, pl.ds(c * L, L)] = gv + jnp.maximum(t * wv[c] + bv[c], 0.0)

        pltpu.async_copy(g, acc_sh.at[db.at[0]], ssem, add=True)
        pltpu.async_copy(ones_v, cnt_sh.at[db.at[0]], ssem, add=True)

    _prep_pk(0, S[0])
    _prep_pk(1, S[1])
    _prep_pk(2, S[2])
    _prep_gather(0, S[0])
    _prep_gather(1, S[1])

    def _step(st, _):
        for b in range(3):
            j = 3 * st + b
            nxt = S[(b + 2) % 3]
            cur = S[b]

            @pl.when(j >= 1)
            def _():
                _drain(nxt)

            @pl.when(j + 2 <= NCH - 1)
            def _():
                _prep_gather(j + 2, nxt)

            @pl.when(j + 3 <= NCH - 1)
            def _():
                _prep_pk(j + 3, cur)

            @pl.when(j <= NCH - 1)
            def _():
                _process(cur)
        return 0
    lax.fori_loop(0, (NCH + 3) // 3, _step, 0)

    plsc.subcore_barrier()

    for k in range(RPT // KE):
        pltpu.sync_copy(acc_sh.at[pl.ds(row0 + k * KE, KE)], g0)
        pltpu.sync_copy(g0, p_hbm.at[cid, pl.ds(row0 + k * KE, KE)])
    pltpu.sync_copy(cnt_sh.at[pl.ds(row0, RPT)], cnt_v)
    pltpu.sync_copy(cnt_v, cnt_hbm.at[cid, pl.ds(row0, RPT)])


def _tc_r():
    BN = 1024

    def body(h_ref, wr_ref, bl_ref, o_ref):
        o_ref[...] = (jnp.dot(h_ref[...], wr_ref[...],
                              preferred_element_type=jnp.float32) + bl_ref[...])

    return pl.pallas_call(
        body,
        grid=(NP // BN,),
        in_specs=[
            pl.BlockSpec((BN, C), lambda i: (i, 0)),
            pl.BlockSpec((C, C), lambda i: (0, 0)),
            pl.BlockSpec((1, C), lambda i: (0, 0)),
        ],
        out_specs=pl.BlockSpec((BN, C), lambda i: (i, 0)),
        out_shape=jax.ShapeDtypeStruct((NP, C), jnp.float32),
    )


def _tc_post(relu_out):
    BN = 1024

    def body(p_ref, cnt_ref, hr_ref, wl_ref, o_ref):
        summ = p_ref[0] + p_ref[1]
        c = cnt_ref[:, 0:1] + cnt_ref[:, 1:2]
        agg = summ / jnp.maximum(c, 1.0)
        out = (jnp.dot(agg, wl_ref[...], preferred_element_type=jnp.float32)
               + hr_ref[...])
        nrm = jnp.sqrt(jnp.sum(out * out, axis=-1, keepdims=True))
        out = out / jnp.maximum(nrm, 1e-12)
        if relu_out:
            out = jnp.maximum(out, 0.0)
        o_ref[...] = out

    return pl.pallas_call(
        body,
        grid=(NP // BN,),
        in_specs=[
            pl.BlockSpec((2, BN, C), lambda i: (0, i, 0)),
            pl.BlockSpec((BN, 2), lambda i: (i, 0)),
            pl.BlockSpec((BN, C), lambda i: (i, 0)),
            pl.BlockSpec((C, C), lambda i: (0, 0)),
        ],
        out_specs=pl.BlockSpec((BN, C), lambda i: (i, 0)),
        out_shape=jax.ShapeDtypeStruct((NP, C), jnp.float32),
    )


_tc_right = _tc_r()
_tc_layer = _tc_post(True)
_tc_final = _tc_post(False)


def kernel(x, edge_index, spd, W_e0, b_e0, W_l0, b_l0, W_r0,
           W_e1, b_e1, W_l1, b_l1, W_r1, W_e2, b_e2, W_l2, b_l2, W_r2):
    s = jnp.mean(spd, axis=1)
    s2 = jnp.broadcast_to(s[:, None], (N, L))
    packed = (edge_index[0] | (edge_index[1] << 14)).reshape(NW, NCH, KE)

    h = jnp.pad(x, ((0, NP - N), (0, 0)))
    layers = ((W_e0, b_e0, W_l0, b_l0, W_r0, True),
              (W_e1, b_e1, W_l1, b_l1, W_r1, True),
              (W_e2, b_e2, W_l2, b_l2, W_r2, False))
    for We, be, Wl, bl, Wr, inner in layers:
        p, cnt = _sc_agg(h, packed, s2, We[:, 0], 2.0 * be)
        hr = _tc_right(h, Wr.T, bl[None])
        tc = _tc_layer if inner else _tc_final
        h = tc(p, cnt.T, hr, Wl.T)
    return h[:N]
